# baseline (speedup 1.0000x reference)
import jax
import jax.numpy as jnp
from jax.experimental import pallas as pl
from jax.experimental.pallas import tpu as pltpu

_BR = 200
_NBUF = 3


def _mega_kernel(
    adj1_ref, adj2_ref, x_ref, w1_ref, b1_ref, w2_ref, b2_ref,
    out_ref, x_vmem, s_ref, g_ref, x_sem,
):
    n = s_ref.shape[0]
    nb = n // _BR

    cp_x = pltpu.make_async_copy(x_ref, x_vmem, x_sem)
    cp_x.start()

    def body(adj1_blk, adj2_blk):
        i = pl.program_id(0)

        @pl.when(i == 0)
        def _():
            cp_x.wait()
            s_ref[...] = jnp.dot(
                x_vmem[...], w1_ref[...], preferred_element_type=jnp.float32
            )

        @pl.when(i < nb)
        def _():
            h = (
                jnp.dot(
                    adj1_blk[...], s_ref[...], preferred_element_type=jnp.float32
                )
                + b1_ref[...]
            )
            h = jnp.maximum(h, 0.0)
            g_ref[pl.ds(i * _BR, _BR), :] = jnp.dot(
                h, w2_ref[...], preferred_element_type=jnp.float32
            )

        @pl.when(i >= nb)
        def _():
            j = i - nb
            logits = (
                jnp.dot(
                    adj2_blk[...], g_ref[...], preferred_element_type=jnp.float32
                )
                + b2_ref[...]
            )
            m = jnp.max(logits, axis=1, keepdims=True)
            lse = m + jnp.log(jnp.sum(jnp.exp(logits - m), axis=1, keepdims=True))
            out_ref[pl.ds(j * _BR, _BR), :] = logits - lse

    pltpu.emit_pipeline(
        body,
        grid=(2 * nb,),
        in_specs=[
            pl.BlockSpec(
                (_BR, n), lambda i: (jnp.minimum(i, nb - 1), 0),
                pipeline_mode=pl.Buffered(buffer_count=2),
            ),
            pl.BlockSpec(
                (_BR, n), lambda i: (jnp.maximum(i - nb, 0), 0),
                pipeline_mode=pl.Buffered(buffer_count=_NBUF, use_lookahead=True),
            ),
        ],
    )(adj1_ref, adj2_ref)


@jax.jit
def kernel(x, adj_1, adj_2, W1, b1, W2, b2):
    n, nfeat = x.shape
    nhid = W1.shape[1]
    nclass = W2.shape[1]
    b1_2d = b1.reshape(1, nhid)
    b2_2d = b2.reshape(1, nclass)

    out = pl.pallas_call(
        _mega_kernel,
        in_specs=[
            pl.BlockSpec(memory_space=pltpu.HBM),
            pl.BlockSpec(memory_space=pltpu.HBM),
            pl.BlockSpec(memory_space=pltpu.HBM),
            pl.BlockSpec(memory_space=pltpu.VMEM),
            pl.BlockSpec(memory_space=pltpu.VMEM),
            pl.BlockSpec(memory_space=pltpu.VMEM),
            pl.BlockSpec(memory_space=pltpu.VMEM),
        ],
        out_specs=pl.BlockSpec(memory_space=pltpu.VMEM),
        out_shape=jax.ShapeDtypeStruct((n, nclass), jnp.float32),
        scratch_shapes=[
            pltpu.VMEM((n, nfeat), jnp.float32),
            pltpu.VMEM((n, nhid), jnp.float32),
            pltpu.VMEM((n, nclass), jnp.float32),
            pltpu.SemaphoreType.DMA,
        ],
    )(adj_1, adj_2, x, W1, b1_2d, W2, b2_2d)

    return out


# speedup vs baseline: 1.0082x; 1.0082x over previous
import jax
import jax.numpy as jnp
from jax.experimental import pallas as pl
from jax.experimental.pallas import tpu as pltpu

_BR = 200
_NBUF = 4


def _mega_kernel(
    adj1_ref, adj2_ref, x_ref, w1_ref, b1_ref, w2_ref, b2_ref,
    out_ref, x_vmem, s_ref, g_ref, x_sem,
):
    n = s_ref.shape[0]

    cp_x = pltpu.make_async_copy(x_ref, x_vmem, x_sem)
    cp_x.start()

    def body1(adj_blk):
        i = pl.program_id(0)

        @pl.when(i == 0)
        def _():
            cp_x.wait()
            s_ref[...] = jnp.dot(
                x_vmem[...], w1_ref[...], preferred_element_type=jnp.float32
            )

        h = (
            jnp.dot(adj_blk[...], s_ref[...], preferred_element_type=jnp.float32)
            + b1_ref[...]
        )
        h = jnp.maximum(h, 0.0)
        g_ref[pl.ds(i * _BR, _BR), :] = jnp.dot(
            h, w2_ref[...], preferred_element_type=jnp.float32
        )

    pltpu.emit_pipeline(
        body1,
        grid=(n // _BR,),
        in_specs=[
            pl.BlockSpec(
                (_BR, n), lambda i: (i, 0),
                pipeline_mode=pl.Buffered(buffer_count=_NBUF),
            )
        ],
    )(adj1_ref)

    def body2(adj_blk):
        i = pl.program_id(0)
        logits = (
            jnp.dot(adj_blk[...], g_ref[...], preferred_element_type=jnp.float32)
            + b2_ref[...]
        )
        m = jnp.max(logits, axis=1, keepdims=True)
        lse = m + jnp.log(jnp.sum(jnp.exp(logits - m), axis=1, keepdims=True))
        out_ref[pl.ds(i * _BR, _BR), :] = logits - lse

    pltpu.emit_pipeline(
        body2,
        grid=(n // _BR,),
        in_specs=[
            pl.BlockSpec(
                (_BR, n), lambda i: (i, 0),
                pipeline_mode=pl.Buffered(buffer_count=_NBUF),
            )
        ],
    )(adj2_ref)


@jax.jit
def kernel(x, adj_1, adj_2, W1, b1, W2, b2):
    n, nfeat = x.shape
    nhid = W1.shape[1]
    nclass = W2.shape[1]
    b1_2d = b1.reshape(1, nhid)
    b2_2d = b2.reshape(1, nclass)

    out = pl.pallas_call(
        _mega_kernel,
        in_specs=[
            pl.BlockSpec(memory_space=pltpu.HBM),
            pl.BlockSpec(memory_space=pltpu.HBM),
            pl.BlockSpec(memory_space=pltpu.HBM),
            pl.BlockSpec(memory_space=pltpu.VMEM),
            pl.BlockSpec(memory_space=pltpu.VMEM),
            pl.BlockSpec(memory_space=pltpu.VMEM),
            pl.BlockSpec(memory_space=pltpu.VMEM),
        ],
        out_specs=pl.BlockSpec(memory_space=pltpu.VMEM),
        out_shape=jax.ShapeDtypeStruct((n, nclass), jnp.float32),
        scratch_shapes=[
            pltpu.VMEM((n, nfeat), jnp.float32),
            pltpu.VMEM((n, nhid), jnp.float32),
            pltpu.VMEM((n, nclass), jnp.float32),
            pltpu.SemaphoreType.DMA,
        ],
    )(adj_1, adj_2, x, W1, b1_2d, W2, b2_2d)

    return out
